# Initial kernel scaffold; baseline (speedup 1.0000x reference)
import jax
import jax.numpy as jnp
from jax.experimental import pallas as pl


def kernel(x_nchw, s1, t1, e1, g1, be1, s2, t2, e2, g2, be2, w_fc_rhc, b_fc_pad, w1_hwio, b1, w2_hwio, b2, g1_raw, be1_raw, g2_raw, be2_raw, w_fc, b_fc):
    raise NotImplementedError("write your pallas kernel here")


# speedup vs baseline: 21.6760x; 21.6760x over previous
import functools

import jax
import jax.numpy as jnp
from jax.experimental import pallas as pl
from jax.experimental.pallas import tpu as pltpu

EPS = 1e-5
NUM_CLASSES = 10
H1 = W1 = 28
C1, C2 = 16, 32
H2 = W2 = 14
L1 = W1 + 4
G1 = W2 + 4
M1 = G1 * C1
M2 = (W2 // 2) * C2
PR = 7
K1 = 5 * L1
N1 = 2 * M1
K2 = 5 * M1
N2 = 2 * M2
FC_PAD = 128
FC_K = PR * M2


def _l1_lhs(x4, rb):
    parts = []
    for kh in range(5):
        c = rb + kh
        parts.append(x4[:, c % 4, (c // 4):(c // 4) + PR, :])
    return jnp.concatenate(parts, axis=2)


def _stats1_kernel(x4_ref, s1_ref, o_ref, *, bsz):
    x4 = x4_ref[...]
    s1 = s1_ref[...]
    tot = None
    ssq = None
    for rb in range(4):
        lhs = _l1_lhs(x4, rb).reshape(bsz * PR, K1)
        p = jnp.dot(lhs, s1, preferred_element_type=jnp.float32)
        s = jnp.sum(p, axis=0, keepdims=True)
        q = jnp.sum(p * p, axis=0, keepdims=True)
        tot = s if tot is None else tot + s
        ssq = q if ssq is None else ssq + q
    o_ref[0, 0:1, :] = tot
    o_ref[0, 1:2, :] = ssq


def _fused_kernel(x4_ref, s1_ref, scsh1_ref, s2_ref, raw_ref, st_ref, *, bsz):
    sc1 = scsh1_ref[0:1, :]
    sh1 = scsh1_ref[1:2, :]
    x4 = x4_ref[...]
    s1 = s1_ref[...]
    zrow = jnp.zeros((bsz, 1, M1), jnp.bfloat16)
    slabs = []
    for par in range(2):
        m = None
        for s in range(2):
            lhs = _l1_lhs(x4, 2 * par + s).reshape(bsz * PR, K1)
            p = jnp.dot(lhs, s1, preferred_element_type=jnp.float32)
            y = p * sc1 + sh1
            yq = jnp.maximum(y[:, :M1], y[:, M1:])
            m = yq if m is None else jnp.maximum(m, yq)
        act = jnp.maximum(m, 0.0).astype(jnp.bfloat16).reshape(bsz, PR, M1)
        slabs.append(jnp.concatenate([zrow, act, zrow], axis=1))
    s2 = s2_ref[...]
    tot = None
    ssq = None
    for s in range(2):
        parts = []
        for kh in range(5):
            c = s + kh
            parts.append(slabs[c % 2][:, (c // 2):(c // 2) + PR, :])
        lhs2 = jnp.concatenate(parts, axis=2).reshape(bsz * PR, K2)
        p2 = jnp.dot(lhs2, s2, preferred_element_type=jnp.float32)
        raw_ref[:, s * PR:(s + 1) * PR, :] = (
            p2.astype(jnp.bfloat16).reshape(bsz, PR, N2))
        sm = jnp.sum(p2, axis=0, keepdims=True)
        sq = jnp.sum(p2 * p2, axis=0, keepdims=True)
        tot = sm if tot is None else tot + sm
        ssq = sq if ssq is None else ssq + sq
    st_ref[0, 0:1, :] = tot
    st_ref[0, 1:2, :] = ssq


def _apply2_kernel(raw_ref, scsh2_ref, wfc_ref, bfc_ref, o_ref, *, bsz):
    sc = scsh2_ref[0:1, :].reshape(1, 1, N2)
    sh = scsh2_ref[1:2, :].reshape(1, 1, N2)
    y0 = raw_ref[:, 0:PR, :].astype(jnp.float32) * sc + sh
    y1 = raw_ref[:, PR:2 * PR, :].astype(jnp.float32) * sc + sh
    m = jnp.maximum(y0, y1)
    act = jnp.maximum(jnp.maximum(m[..., :M2], m[..., M2:]), 0.0)
    lhs = jnp.concatenate([act[:, h, :] for h in range(PR)], axis=1)
    w = wfc_ref[...].reshape(FC_K, FC_PAD)
    o_ref[...] = (jnp.dot(lhs, w, preferred_element_type=jnp.float32)
                  + bfc_ref[...])


def _stats1_call(x4, s1c, nblk, bsz):
    return pl.pallas_call(
        functools.partial(_stats1_kernel, bsz=bsz),
        grid=(nblk,),
        in_specs=[
            pl.BlockSpec((bsz, 4, 8, L1), lambda t: (t, 0, 0, 0)),
            pl.BlockSpec((K1, N1), lambda t: (0, 0)),
        ],
        out_specs=pl.BlockSpec((1, 2, N1), lambda t: (t, 0, 0)),
        out_shape=jax.ShapeDtypeStruct((nblk, 2, N1), jnp.float32),
        compiler_params=pltpu.CompilerParams(
            dimension_semantics=("arbitrary",)),
    )(x4, s1c)


def _fused_call(x4, s1c, scsh1, s2c, n, nblk, bsz):
    return pl.pallas_call(
        functools.partial(_fused_kernel, bsz=bsz),
        grid=(nblk,),
        in_specs=[
            pl.BlockSpec((bsz, 4, 8, L1), lambda t: (t, 0, 0, 0)),
            pl.BlockSpec((K1, N1), lambda t: (0, 0)),
            pl.BlockSpec((2, N1), lambda t: (0, 0)),
            pl.BlockSpec((K2, N2), lambda t: (0, 0)),
        ],
        out_specs=[
            pl.BlockSpec((bsz, 2 * PR, N2), lambda t: (t, 0, 0)),
            pl.BlockSpec((1, 2, N2), lambda t: (t, 0, 0)),
        ],
        out_shape=[
            jax.ShapeDtypeStruct((n, 2 * PR, N2), jnp.bfloat16),
            jax.ShapeDtypeStruct((nblk, 2, N2), jnp.float32),
        ],
        compiler_params=pltpu.CompilerParams(
            dimension_semantics=("arbitrary",)),
    )(x4, s1c, scsh1, s2c)


def _apply2_call(raw2, scsh2, wfc, bfc, n, nblk, bsz):
    return pl.pallas_call(
        functools.partial(_apply2_kernel, bsz=bsz),
        grid=(nblk,),
        in_specs=[
            pl.BlockSpec((bsz, 2 * PR, N2), lambda t: (t, 0, 0)),
            pl.BlockSpec((2, N2), lambda t: (0, 0)),
            pl.BlockSpec((PR, M2, FC_PAD), lambda t: (0, 0, 0)),
            pl.BlockSpec((1, FC_PAD), lambda t: (0, 0)),
        ],
        out_specs=pl.BlockSpec((bsz, FC_PAD), lambda t: (t, 0)),
        out_shape=jax.ShapeDtypeStruct((n, FC_PAD), jnp.float32),
        compiler_params=pltpu.CompilerParams(
            dimension_semantics=("arbitrary",)),
    )(raw2, scsh2, wfc, bfc)


def _fold_bn(S, t, e, g, be, inv_count, lanes_half):
    sums = (S[:, :lanes_half] + S[:, lanes_half:]) @ t
    mean = sums[0:1] * inv_count
    var = sums[1:2] * inv_count - mean * mean
    scale = g * jax.lax.rsqrt(var + EPS)
    shift = be - scale * mean
    scsh = jnp.concatenate([scale, shift], axis=0) @ e
    return jnp.tile(scsh, (1, 2))


def kernel(x_nchw, s1, t1, e1, g1, be1, s2, t2, e2, g2, be2, w_fc_rhc,
           b_fc_pad, w1_hwio, b1, w2_hwio, b2, g1_raw, be1_raw, g2_raw,
           be2_raw, w_fc, b_fc):
    n = x_nchw.shape[0]
    bsz = 256
    while n % bsz:
        bsz //= 2
    nblk = n // bsz

    x = x_nchw.reshape(n, H1, W1).astype(jnp.bfloat16)
    xp = jnp.pad(x, ((0, 0), (2, 2), (2, 2)))
    x4 = xp.reshape(n, 8, 4, L1).transpose(0, 2, 1, 3)

    s1c = s1.transpose(0, 2, 1, 3).reshape(K1, N1).astype(jnp.bfloat16)
    s2c = s2.transpose(0, 2, 1, 3).reshape(K2, N2).astype(jnp.bfloat16)

    st1 = _stats1_call(x4, s1c, nblk, bsz)
    scsh1 = _fold_bn(jnp.sum(st1, axis=0), t1, e1, g1, be1,
                     1.0 / (n * H1 * W1), M1)

    raw2, st2 = _fused_call(x4, s1c, scsh1, s2c, n, nblk, bsz)
    scsh2 = _fold_bn(jnp.sum(st2, axis=0), t2, e2, g2, be2,
                     1.0 / (n * H2 * W2), M2)

    logits = _apply2_call(raw2, scsh2, w_fc_rhc, b_fc_pad, n, nblk, bsz)
    return logits[:, :NUM_CLASSES]


# speedup vs baseline: 22.4856x; 1.0373x over previous
import functools

import jax
import jax.numpy as jnp
from jax.experimental import pallas as pl
from jax.experimental.pallas import tpu as pltpu

EPS = 1e-5
NUM_CLASSES = 10
H1 = W1 = 28
C1, C2 = 16, 32
H2 = W2 = 14
L1 = W1 + 4
G1 = W2 + 4
M1 = G1 * C1
M2 = (W2 // 2) * C2
PR = 7
K1 = 5 * L1
N1 = 2 * M1
K2 = 5 * M1
N2 = 2 * M2
FC_PAD = 128
FC_K = PR * M2


def _l1_lhs(x4, rb):
    parts = []
    for kh in range(5):
        c = rb + kh
        parts.append(x4[:, c % 4, (c // 4):(c // 4) + PR, :])
    return jnp.concatenate(parts, axis=2)


def _conv1_kernel(x4_ref, s1_ref, mm_ref, st_ref, *, bsz):
    x4 = x4_ref[...]
    s1 = s1_ref[...]
    tot = None
    ssq = None
    for par in range(2):
        pmax = None
        pmin = None
        for s in range(2):
            lhs = _l1_lhs(x4, 2 * par + s).reshape(bsz * PR, K1)
            p = jnp.dot(lhs, s1, preferred_element_type=jnp.float32)
            sm = jnp.sum(p, axis=0, keepdims=True)
            sq = jnp.sum(p * p, axis=0, keepdims=True)
            tot = sm if tot is None else tot + sm
            ssq = sq if ssq is None else ssq + sq
            hi = jnp.maximum(p[:, :M1], p[:, M1:])
            lo = jnp.minimum(p[:, :M1], p[:, M1:])
            pmax = hi if pmax is None else jnp.maximum(pmax, hi)
            pmin = lo if pmin is None else jnp.minimum(pmin, lo)
        mm_ref[:, 2 * par, :, :] = (
            pmax.astype(jnp.bfloat16).reshape(bsz, PR, M1))
        mm_ref[:, 2 * par + 1, :, :] = (
            pmin.astype(jnp.bfloat16).reshape(bsz, PR, M1))
    st_ref[0, 0:1, :] = tot
    st_ref[0, 1:2, :] = ssq


def _conv2_kernel(mm_ref, scsh1_ref, s2_ref, mm2_ref, st_ref, *, bsz):
    scp = scsh1_ref[0:1, :].reshape(1, 1, M1)
    scn = scsh1_ref[1:2, :].reshape(1, 1, M1)
    sh = scsh1_ref[2:3, :].reshape(1, 1, M1)
    zrow = jnp.zeros((bsz, 1, M1), jnp.bfloat16)
    slabs = []
    for par in range(2):
        pmax = mm_ref[:, 2 * par, :, :].astype(jnp.float32)
        pmin = mm_ref[:, 2 * par + 1, :, :].astype(jnp.float32)
        act = jnp.maximum(scp * pmax + scn * pmin + sh, 0.0)
        act = act.astype(jnp.bfloat16)
        slabs.append(jnp.concatenate([zrow, act, zrow], axis=1))
    s2 = s2_ref[...]
    tot = None
    ssq = None
    pmax2 = None
    pmin2 = None
    for s in range(2):
        parts = []
        for kh in range(5):
            c = s + kh
            parts.append(slabs[c % 2][:, (c // 2):(c // 2) + PR, :])
        lhs2 = jnp.concatenate(parts, axis=2).reshape(bsz * PR, K2)
        p2 = jnp.dot(lhs2, s2, preferred_element_type=jnp.float32)
        sm = jnp.sum(p2, axis=0, keepdims=True)
        sq = jnp.sum(p2 * p2, axis=0, keepdims=True)
        tot = sm if tot is None else tot + sm
        ssq = sq if ssq is None else ssq + sq
        hi = jnp.maximum(p2[:, :M2], p2[:, M2:])
        lo = jnp.minimum(p2[:, :M2], p2[:, M2:])
        pmax2 = hi if pmax2 is None else jnp.maximum(pmax2, hi)
        pmin2 = lo if pmin2 is None else jnp.minimum(pmin2, lo)
    mm2_ref[:, 0, :, :] = pmax2.astype(jnp.bfloat16).reshape(bsz, PR, M2)
    mm2_ref[:, 1, :, :] = pmin2.astype(jnp.bfloat16).reshape(bsz, PR, M2)
    st_ref[0, 0:1, :] = tot
    st_ref[0, 1:2, :] = ssq


def _apply2_kernel(mm2_ref, scsh2_ref, wfc_ref, bfc_ref, o_ref, *, bsz):
    scp = scsh2_ref[0:1, :].reshape(1, 1, M2)
    scn = scsh2_ref[1:2, :].reshape(1, 1, M2)
    sh = scsh2_ref[2:3, :].reshape(1, 1, M2)
    pmax = mm2_ref[:, 0, :, :].astype(jnp.float32)
    pmin = mm2_ref[:, 1, :, :].astype(jnp.float32)
    act = jnp.maximum(scp * pmax + scn * pmin + sh, 0.0)
    lhs = jnp.concatenate([act[:, h, :] for h in range(PR)], axis=1)
    w = wfc_ref[...].reshape(FC_K, FC_PAD)
    o_ref[...] = (jnp.dot(lhs, w, preferred_element_type=jnp.float32)
                  + bfc_ref[...])


def _conv1_call(x4, s1c, n, nblk, bsz):
    return pl.pallas_call(
        functools.partial(_conv1_kernel, bsz=bsz),
        grid=(nblk,),
        in_specs=[
            pl.BlockSpec((bsz, 4, 8, L1), lambda t: (t, 0, 0, 0)),
            pl.BlockSpec((K1, N1), lambda t: (0, 0)),
        ],
        out_specs=[
            pl.BlockSpec((bsz, 4, PR, M1), lambda t: (t, 0, 0, 0)),
            pl.BlockSpec((1, 2, N1), lambda t: (t, 0, 0)),
        ],
        out_shape=[
            jax.ShapeDtypeStruct((n, 4, PR, M1), jnp.bfloat16),
            jax.ShapeDtypeStruct((nblk, 2, N1), jnp.float32),
        ],
        compiler_params=pltpu.CompilerParams(
            dimension_semantics=("arbitrary",)),
    )(x4, s1c)


def _conv2_call(mm1, scsh1, s2c, n, nblk, bsz):
    return pl.pallas_call(
        functools.partial(_conv2_kernel, bsz=bsz),
        grid=(nblk,),
        in_specs=[
            pl.BlockSpec((bsz, 4, PR, M1), lambda t: (t, 0, 0, 0)),
            pl.BlockSpec((3, M1), lambda t: (0, 0)),
            pl.BlockSpec((K2, N2), lambda t: (0, 0)),
        ],
        out_specs=[
            pl.BlockSpec((bsz, 2, PR, M2), lambda t: (t, 0, 0, 0)),
            pl.BlockSpec((1, 2, N2), lambda t: (t, 0, 0)),
        ],
        out_shape=[
            jax.ShapeDtypeStruct((n, 2, PR, M2), jnp.bfloat16),
            jax.ShapeDtypeStruct((nblk, 2, N2), jnp.float32),
        ],
        compiler_params=pltpu.CompilerParams(
            dimension_semantics=("arbitrary",)),
    )(mm1, scsh1, s2c)


def _apply2_call(mm2, scsh2, wfc, bfc, n, nblk, bsz):
    return pl.pallas_call(
        functools.partial(_apply2_kernel, bsz=bsz),
        grid=(nblk,),
        in_specs=[
            pl.BlockSpec((bsz, 2, PR, M2), lambda t: (t, 0, 0, 0)),
            pl.BlockSpec((3, M2), lambda t: (0, 0)),
            pl.BlockSpec((PR, M2, FC_PAD), lambda t: (0, 0, 0)),
            pl.BlockSpec((1, FC_PAD), lambda t: (0, 0)),
        ],
        out_specs=pl.BlockSpec((bsz, FC_PAD), lambda t: (t, 0)),
        out_shape=jax.ShapeDtypeStruct((n, FC_PAD), jnp.float32),
        compiler_params=pltpu.CompilerParams(
            dimension_semantics=("arbitrary",)),
    )(mm2, scsh2, wfc, bfc)


def _fold_bn(S, t, e, g, be, inv_count, lanes_half):
    sums = (S[:, :lanes_half] + S[:, lanes_half:]) @ t
    mean = sums[0:1] * inv_count
    var = sums[1:2] * inv_count - mean * mean
    scale = g * jax.lax.rsqrt(var + EPS)
    shift = be - scale * mean
    trip = jnp.concatenate(
        [jnp.maximum(scale, 0.0), jnp.minimum(scale, 0.0), shift], axis=0)
    return trip @ e


def kernel(x_nchw, s1, t1, e1, g1, be1, s2, t2, e2, g2, be2, w_fc_rhc,
           b_fc_pad, w1_hwio, b1, w2_hwio, b2, g1_raw, be1_raw, g2_raw,
           be2_raw, w_fc, b_fc):
    n = x_nchw.shape[0]
    bsz = 256
    while n % bsz:
        bsz //= 2
    nblk = n // bsz

    x = x_nchw.reshape(n, H1, W1).astype(jnp.bfloat16)
    xp = jnp.pad(x, ((0, 0), (2, 2), (2, 2)))
    x4 = xp.reshape(n, 8, 4, L1).transpose(0, 2, 1, 3)

    s1c = s1.transpose(0, 2, 1, 3).reshape(K1, N1).astype(jnp.bfloat16)
    s2c = s2.transpose(0, 2, 1, 3).reshape(K2, N2).astype(jnp.bfloat16)

    mm1, st1 = _conv1_call(x4, s1c, n, nblk, bsz)
    scsh1 = _fold_bn(jnp.sum(st1, axis=0), t1, e1, g1, be1,
                     1.0 / (n * H1 * W1), M1)

    mm2, st2 = _conv2_call(mm1, scsh1, s2c, n, nblk, bsz)
    scsh2 = _fold_bn(jnp.sum(st2, axis=0), t2, e2, g2, be2,
                     1.0 / (n * H2 * W2), M2)

    logits = _apply2_call(mm2, scsh2, w_fc_rhc, b_fc_pad, n, nblk, bsz)
    return logits[:, :NUM_CLASSES]


# speedup vs baseline: 34.3373x; 1.5271x over previous
import functools

import jax
import jax.numpy as jnp
from jax.experimental import pallas as pl
from jax.experimental.pallas import tpu as pltpu

EPS = 1e-5
NUM_CLASSES = 10
H1 = W1 = 28
C1, C2 = 16, 32
H2 = W2 = 14
PR = 7
LX = 128
K1 = 2 * LX
CH1 = 256
N1 = 8 * CH1
CH2 = 256
K2 = 6 * CH2
CQ2 = 256
N2 = 4 * CQ2
FC_PAD = 128
FC_K = PR * CQ2


def _conv1_kernel(xl_ref, s1_ref, mm_ref, g_ref, rs_ref, *, bsz, nblk):
    t = pl.program_id(0)
    xl = xl_ref[...]
    lhs = jnp.concatenate([xl[:, 0:PR, :], xl[:, 1:PR + 1, :]],
                          axis=2).reshape(bsz * PR, K1)
    p = jnp.dot(lhs, s1_ref[...], preferred_element_type=jnp.float32)
    gram = jax.lax.dot_general(lhs, lhs, (((0,), (0,)), ((), ())),
                               preferred_element_type=jnp.float32)
    rsum = jnp.sum(lhs.astype(jnp.float32), axis=0, keepdims=True)

    @pl.when(t == 0)
    def _init():
        g_ref[...] = jnp.zeros_like(g_ref)
        rs_ref[...] = jnp.zeros_like(rs_ref)

    g_ref[...] += gram
    rs_ref[...] += rsum

    for par in range(2):
        c0 = p[:, (4 * par + 0) * CH1:(4 * par + 1) * CH1]
        c1 = p[:, (4 * par + 1) * CH1:(4 * par + 2) * CH1]
        c2 = p[:, (4 * par + 2) * CH1:(4 * par + 3) * CH1]
        c3 = p[:, (4 * par + 3) * CH1:(4 * par + 4) * CH1]
        pmax = jnp.maximum(jnp.maximum(c0, c1), jnp.maximum(c2, c3))
        pmin = jnp.minimum(jnp.minimum(c0, c1), jnp.minimum(c2, c3))
        mm_ref[:, 2 * par, :, :] = (
            pmax.astype(jnp.bfloat16).reshape(bsz, PR, CH1))
        mm_ref[:, 2 * par + 1, :, :] = (
            pmin.astype(jnp.bfloat16).reshape(bsz, PR, CH1))


def _conv2_kernel(mm_ref, scsh1_ref, s2_ref, mm2_ref, st_ref, *, bsz, nblk):
    t = pl.program_id(0)
    scp = scsh1_ref[0:1, :].reshape(1, 1, CH2)
    scn = scsh1_ref[1:2, :].reshape(1, 1, CH2)
    sh = scsh1_ref[2:3, :].reshape(1, 1, CH2)
    acts = []
    for par in range(2):
        pmax = mm_ref[:, 2 * par, :, :].astype(jnp.float32)
        pmin = mm_ref[:, 2 * par + 1, :, :].astype(jnp.float32)
        act = jnp.maximum(scp * pmax + scn * pmin + sh, 0.0)
        acts.append(act.astype(jnp.bfloat16))
    zrow = jnp.zeros((bsz, 1, 2 * CH2), jnp.bfloat16)
    eo = jnp.concatenate(
        [zrow, jnp.concatenate(acts, axis=2), zrow], axis=1)
    lhs2 = jnp.concatenate(
        [eo[:, 0:PR, :], eo[:, 1:PR + 1, :], eo[:, 2:PR + 2, :]],
        axis=2).reshape(bsz * PR, K2)
    p2 = jnp.dot(lhs2, s2_ref[...], preferred_element_type=jnp.float32)

    tot = jnp.sum(p2, axis=0, keepdims=True)
    ssq = jnp.sum(p2 * p2, axis=0, keepdims=True)

    @pl.when(t == 0)
    def _init():
        st_ref[...] = jnp.zeros_like(st_ref)

    st_ref[0:1, :] += tot
    st_ref[1:2, :] += ssq

    c0 = p2[:, 0 * CQ2:1 * CQ2]
    c1 = p2[:, 1 * CQ2:2 * CQ2]
    c2 = p2[:, 2 * CQ2:3 * CQ2]
    c3 = p2[:, 3 * CQ2:4 * CQ2]
    pmax2 = jnp.maximum(jnp.maximum(c0, c1), jnp.maximum(c2, c3))
    pmin2 = jnp.minimum(jnp.minimum(c0, c1), jnp.minimum(c2, c3))
    mm2_ref[:, 0, :, :] = pmax2.astype(jnp.bfloat16).reshape(bsz, PR, CQ2)
    mm2_ref[:, 1, :, :] = pmin2.astype(jnp.bfloat16).reshape(bsz, PR, CQ2)


def _apply2_kernel(mm2_ref, scsh2_ref, wfc_ref, bfc_ref, o_ref, *, bsz):
    scp = scsh2_ref[0:1, :].reshape(1, 1, CQ2)
    scn = scsh2_ref[1:2, :].reshape(1, 1, CQ2)
    sh = scsh2_ref[2:3, :].reshape(1, 1, CQ2)
    pmax = mm2_ref[:, 0, :, :].astype(jnp.float32)
    pmin = mm2_ref[:, 1, :, :].astype(jnp.float32)
    act = jnp.maximum(scp * pmax + scn * pmin + sh, 0.0)
    lhs = jnp.concatenate([act[:, h, :] for h in range(PR)], axis=1)
    o_ref[...] = (jnp.dot(lhs, wfc_ref[...],
                          preferred_element_type=jnp.float32)
                  + bfc_ref[...])


def _conv1_call(xl, s1w, n, nblk, bsz):
    return pl.pallas_call(
        functools.partial(_conv1_kernel, bsz=bsz, nblk=nblk),
        grid=(nblk,),
        in_specs=[
            pl.BlockSpec((bsz, 8, LX), lambda t: (t, 0, 0)),
            pl.BlockSpec((K1, N1), lambda t: (0, 0)),
        ],
        out_specs=[
            pl.BlockSpec((bsz, 4, PR, CH1), lambda t: (t, 0, 0, 0)),
            pl.BlockSpec((K1, K1), lambda t: (0, 0)),
            pl.BlockSpec((1, K1), lambda t: (0, 0)),
        ],
        out_shape=[
            jax.ShapeDtypeStruct((n, 4, PR, CH1), jnp.bfloat16),
            jax.ShapeDtypeStruct((K1, K1), jnp.float32),
            jax.ShapeDtypeStruct((1, K1), jnp.float32),
        ],
        compiler_params=pltpu.CompilerParams(
            dimension_semantics=("arbitrary",)),
    )(xl, s1w)


def _conv2_call(mm1, scsh1, s2w, n, nblk, bsz):
    return pl.pallas_call(
        functools.partial(_conv2_kernel, bsz=bsz, nblk=nblk),
        grid=(nblk,),
        in_specs=[
            pl.BlockSpec((bsz, 4, PR, CH1), lambda t: (t, 0, 0, 0)),
            pl.BlockSpec((3, CH2), lambda t: (0, 0)),
            pl.BlockSpec((K2, N2), lambda t: (0, 0)),
        ],
        out_specs=[
            pl.BlockSpec((bsz, 2, PR, CQ2), lambda t: (t, 0, 0, 0)),
            pl.BlockSpec((2, N2), lambda t: (0, 0)),
        ],
        out_shape=[
            jax.ShapeDtypeStruct((n, 2, PR, CQ2), jnp.bfloat16),
            jax.ShapeDtypeStruct((2, N2), jnp.float32),
        ],
        compiler_params=pltpu.CompilerParams(
            dimension_semantics=("arbitrary",)),
    )(mm1, scsh1, s2w)


def _apply2_call(mm2, scsh2, wfc, bfc, n, nblk, bsz):
    return pl.pallas_call(
        functools.partial(_apply2_kernel, bsz=bsz),
        grid=(nblk,),
        in_specs=[
            pl.BlockSpec((bsz, 2, PR, CQ2), lambda t: (t, 0, 0, 0)),
            pl.BlockSpec((3, CQ2), lambda t: (0, 0)),
            pl.BlockSpec((FC_K, FC_PAD), lambda t: (0, 0)),
            pl.BlockSpec((1, FC_PAD), lambda t: (0, 0)),
        ],
        out_specs=pl.BlockSpec((bsz, FC_PAD), lambda t: (t, 0)),
        out_shape=jax.ShapeDtypeStruct((n, FC_PAD), jnp.float32),
        compiler_params=pltpu.CompilerParams(
            dimension_semantics=("arbitrary",)),
    )(mm2, scsh2, wfc, bfc)


def _minmax_trip(scale, shift, reps, lanes):
    trip = jnp.concatenate(
        [jnp.maximum(scale, 0.0), jnp.minimum(scale, 0.0), shift], axis=0)
    tiled = jnp.tile(trip, (1, reps))
    pad = lanes - tiled.shape[1]
    return jnp.pad(tiled, ((0, 0), (0, pad)))


def kernel(x_nchw, s1, t1, e1, g1, be1, s2, t2, e2, g2, be2, w_fc_rhc,
           b_fc_pad, w1_hwio, b1, w2_hwio, b2, g1_raw, be1_raw, g2_raw,
           be2_raw, w_fc, b_fc):
    n = x_nchw.shape[0]
    bsz = 256
    while n % bsz:
        bsz //= 2
    nblk = n // bsz
    f32 = jnp.float32

    x = x_nchw.reshape(n, H1, W1).astype(jnp.bfloat16)
    xp = jnp.pad(x, ((0, 0), (2, 2), (2, 2)))
    xl = xp.reshape(n, 8, LX)

    s1r = s1.transpose(0, 2, 1, 3).reshape(5, 32, 2, 18, C1)[:, :, :, 2:16, :]
    s1r = jnp.pad(s1r.reshape(5, 32, 2, 14 * C1),
                  ((0, 0), (0, 0), (0, 0), (0, CH1 - 14 * C1)))
    s1r = s1r.reshape(5 * 32, 2 * CH1)
    zrb = [jnp.zeros((32 * k, 2 * CH1), f32) for k in range(4)]
    s1wf = jnp.concatenate(
        [jnp.concatenate([zrb[rb], s1r, zrb[3 - rb]], axis=0)
         for rb in range(4)], axis=1)
    s1w = s1wf.astype(jnp.bfloat16)

    s2r = s2.transpose(0, 2, 1, 3)
    s2r = s2r.reshape(5, 18, C1, 2, 224)[:, 2:16]
    s2r = jnp.pad(s2r.reshape(5, 14 * C1, 2, 224),
                  ((0, 0), (0, CH2 - 14 * C1), (0, 0), (0, CQ2 - 224)))
    s2r = s2r.reshape(5 * CH2, 2 * CQ2)
    zs = jnp.zeros((CH2, 2 * CQ2), f32)
    s2w = jnp.concatenate(
        [jnp.concatenate([s2r, zs], axis=0),
         jnp.concatenate([zs, s2r], axis=0)], axis=1).astype(jnp.bfloat16)

    wfc = jnp.pad(w_fc_rhc, ((0, 0), (0, CQ2 - 224), (0, 0)))
    wfc = wfc.reshape(FC_K, FC_PAD)

    mm1, gram, rsum = _conv1_call(xl, s1w, n, nblk, bsz)

    s_lanes = (rsum @ s1wf).reshape(8, CH1).sum(axis=0)
    q_lanes = (s1wf * (gram @ s1wf)).sum(axis=0).reshape(8, CH1).sum(axis=0)
    s_ch = s_lanes.reshape(16, C1).sum(axis=0).reshape(1, C1)
    q_ch = q_lanes.reshape(16, C1).sum(axis=0).reshape(1, C1)
    inv1 = 1.0 / (n * H1 * W1)
    mean1 = s_ch * inv1
    var1 = q_ch * inv1 - mean1 * mean1
    scale1 = g1 * jax.lax.rsqrt(var1 + EPS)
    shift1 = be1 - scale1 * mean1
    scsh1 = _minmax_trip(scale1, shift1, 16, CH2)

    mm2, st2 = _conv2_call(mm1, scsh1, s2w, n, nblk, bsz)

    S2 = (st2[:, 0:CQ2] + st2[:, CQ2:2 * CQ2]
          + st2[:, 2 * CQ2:3 * CQ2] + st2[:, 3 * CQ2:])
    ch2 = S2.reshape(2, 8, C2).sum(axis=1)
    inv2 = 1.0 / (n * H2 * W2)
    mean2 = ch2[0:1] * inv2
    var2 = ch2[1:2] * inv2 - mean2 * mean2
    scale2 = g2 * jax.lax.rsqrt(var2 + EPS)
    shift2 = be2 - scale2 * mean2
    scsh2 = _minmax_trip(scale2, shift2, 8, CQ2)

    logits = _apply2_call(mm2, scsh2, wfc, b_fc_pad, n, nblk, bsz)
    return logits[:, :NUM_CLASSES]


# speedup vs baseline: 56.1560x; 1.6354x over previous
import functools

import jax
import jax.numpy as jnp
from jax.experimental import pallas as pl
from jax.experimental.pallas import tpu as pltpu

EPS = 1e-5
NUM_CLASSES = 10
H1 = W1 = 28
C1, C2 = 16, 32
H2 = W2 = 14
PR = 7
LX = 128
K1 = 2 * LX
CH1 = 256
N1 = 8 * CH1
CH2 = 256
K2 = 5 * CH2
CQ2 = 256
N2 = 2 * CQ2
FC_PAD = 128
FC_K = PR * CQ2


def _conv1_kernel(xl_ref, s1_ref, mm_ref, g_ref, rs_ref, *, bsz):
    t = pl.program_id(0)
    xl = xl_ref[...]
    lhs = jnp.concatenate([xl[0:PR], xl[1:PR + 1]],
                          axis=2).reshape(PR * bsz, K1)
    p = jnp.dot(lhs, s1_ref[...], preferred_element_type=jnp.float32)
    gram = jax.lax.dot_general(lhs, lhs, (((0,), (0,)), ((), ())),
                               preferred_element_type=jnp.float32)
    rsum = jnp.sum(lhs.astype(jnp.float32), axis=0, keepdims=True)

    @pl.when(t == 0)
    def _init():
        g_ref[...] = jnp.zeros_like(g_ref)
        rs_ref[...] = jnp.zeros_like(rs_ref)

    g_ref[...] += gram
    rs_ref[...] += rsum

    for par in range(2):
        c0 = p[:, (4 * par + 0) * CH1:(4 * par + 1) * CH1]
        c1 = p[:, (4 * par + 1) * CH1:(4 * par + 2) * CH1]
        c2 = p[:, (4 * par + 2) * CH1:(4 * par + 3) * CH1]
        c3 = p[:, (4 * par + 3) * CH1:(4 * par + 4) * CH1]
        pmax = jnp.maximum(jnp.maximum(c0, c1), jnp.maximum(c2, c3))
        pmin = jnp.minimum(jnp.minimum(c0, c1), jnp.minimum(c2, c3))
        mm_ref[2 * par] = pmax.astype(jnp.bfloat16).reshape(PR, bsz, CH1)
        mm_ref[2 * par + 1] = pmin.astype(jnp.bfloat16).reshape(PR, bsz, CH1)


def _conv2_kernel(mm_ref, scsh1_ref, s2_ref, mm2_ref, st_ref, *, bsz):
    t = pl.program_id(0)
    scp = scsh1_ref[0:1, :].reshape(1, 1, CH2)
    scn = scsh1_ref[1:2, :].reshape(1, 1, CH2)
    sh = scsh1_ref[2:3, :].reshape(1, 1, CH2)
    acts = []
    for par in range(2):
        pmax = mm_ref[2 * par].astype(jnp.float32)
        pmin = mm_ref[2 * par + 1].astype(jnp.float32)
        act = jnp.maximum(scp * pmax + scn * pmin + sh, 0.0)
        acts.append(act.astype(jnp.bfloat16))
    zrow = jnp.zeros((1, bsz, 2 * CH2), jnp.bfloat16)
    eo = jnp.concatenate(
        [zrow, jnp.concatenate(acts, axis=2), zrow], axis=0)
    s2 = s2_ref[...]
    lhs0 = jnp.concatenate(
        [eo[0:PR], eo[1:PR + 1], eo[2:PR + 2, :, 0:CH2]],
        axis=2).reshape(PR * bsz, K2)
    lhs1 = jnp.concatenate(
        [eo[0:PR, :, CH2:], eo[1:PR + 1], eo[2:PR + 2]],
        axis=2).reshape(PR * bsz, K2)
    p0 = jnp.dot(lhs0, s2, preferred_element_type=jnp.float32)
    p1 = jnp.dot(lhs1, s2, preferred_element_type=jnp.float32)

    tot = (jnp.sum(p0, axis=0, keepdims=True)
           + jnp.sum(p1, axis=0, keepdims=True))
    ssq = (jnp.sum(p0 * p0, axis=0, keepdims=True)
           + jnp.sum(p1 * p1, axis=0, keepdims=True))

    @pl.when(t == 0)
    def _init():
        st_ref[...] = jnp.zeros_like(st_ref)

    st_ref[0:1, :] += tot
    st_ref[1:2, :] += ssq

    pmax2 = jnp.maximum(jnp.maximum(p0[:, :CQ2], p0[:, CQ2:]),
                        jnp.maximum(p1[:, :CQ2], p1[:, CQ2:]))
    pmin2 = jnp.minimum(jnp.minimum(p0[:, :CQ2], p0[:, CQ2:]),
                        jnp.minimum(p1[:, :CQ2], p1[:, CQ2:]))
    mm2_ref[0] = pmax2.astype(jnp.bfloat16).reshape(PR, bsz, CQ2)
    mm2_ref[1] = pmin2.astype(jnp.bfloat16).reshape(PR, bsz, CQ2)


def _apply2_kernel(mm2_ref, scsh2_ref, wfc_ref, bfc_ref, o_ref, *, bsz):
    scp = scsh2_ref[0:1, :].reshape(1, 1, CQ2)
    scn = scsh2_ref[1:2, :].reshape(1, 1, CQ2)
    sh = scsh2_ref[2:3, :].reshape(1, 1, CQ2)
    pmax = mm2_ref[0].astype(jnp.float32)
    pmin = mm2_ref[1].astype(jnp.float32)
    act = jnp.maximum(scp * pmax + scn * pmin + sh, 0.0)
    lhs = jnp.concatenate([act[h] for h in range(PR)], axis=1)
    o_ref[...] = (jnp.dot(lhs, wfc_ref[...],
                          preferred_element_type=jnp.float32)
                  + bfc_ref[...])


def _conv1_call(xl, s1w, n, nblk, bsz):
    return pl.pallas_call(
        functools.partial(_conv1_kernel, bsz=bsz),
        grid=(nblk,),
        in_specs=[
            pl.BlockSpec((8, bsz, LX), lambda t: (0, t, 0)),
            pl.BlockSpec((K1, N1), lambda t: (0, 0)),
        ],
        out_specs=[
            pl.BlockSpec((4, PR, bsz, CH1), lambda t: (0, 0, t, 0)),
            pl.BlockSpec((K1, K1), lambda t: (0, 0)),
            pl.BlockSpec((1, K1), lambda t: (0, 0)),
        ],
        out_shape=[
            jax.ShapeDtypeStruct((4, PR, n, CH1), jnp.bfloat16),
            jax.ShapeDtypeStruct((K1, K1), jnp.float32),
            jax.ShapeDtypeStruct((1, K1), jnp.float32),
        ],
        compiler_params=pltpu.CompilerParams(
            dimension_semantics=("arbitrary",)),
    )(xl, s1w)


def _conv2_call(mm1, scsh1, s2w, n, nblk, bsz):
    return pl.pallas_call(
        functools.partial(_conv2_kernel, bsz=bsz),
        grid=(nblk,),
        in_specs=[
            pl.BlockSpec((4, PR, bsz, CH1), lambda t: (0, 0, t, 0)),
            pl.BlockSpec((3, CH2), lambda t: (0, 0)),
            pl.BlockSpec((K2, N2), lambda t: (0, 0)),
        ],
        out_specs=[
            pl.BlockSpec((2, PR, bsz, CQ2), lambda t: (0, 0, t, 0)),
            pl.BlockSpec((2, N2), lambda t: (0, 0)),
        ],
        out_shape=[
            jax.ShapeDtypeStruct((2, PR, n, CQ2), jnp.bfloat16),
            jax.ShapeDtypeStruct((2, N2), jnp.float32),
        ],
        compiler_params=pltpu.CompilerParams(
            dimension_semantics=("arbitrary",)),
    )(mm1, scsh1, s2w)


def _apply2_call(mm2, scsh2, wfc, bfc, n, nblk, bsz):
    return pl.pallas_call(
        functools.partial(_apply2_kernel, bsz=bsz),
        grid=(nblk,),
        in_specs=[
            pl.BlockSpec((2, PR, bsz, CQ2), lambda t: (0, 0, t, 0)),
            pl.BlockSpec((3, CQ2), lambda t: (0, 0)),
            pl.BlockSpec((FC_K, FC_PAD), lambda t: (0, 0)),
            pl.BlockSpec((1, FC_PAD), lambda t: (0, 0)),
        ],
        out_specs=pl.BlockSpec((bsz, FC_PAD), lambda t: (t, 0)),
        out_shape=jax.ShapeDtypeStruct((n, FC_PAD), jnp.float32),
        compiler_params=pltpu.CompilerParams(
            dimension_semantics=("arbitrary",)),
    )(mm2, scsh2, wfc, bfc)


def _minmax_trip(scale, shift, reps, lanes):
    trip = jnp.concatenate(
        [jnp.maximum(scale, 0.0), jnp.minimum(scale, 0.0), shift], axis=0)
    tiled = jnp.tile(trip, (1, reps))
    pad = lanes - tiled.shape[1]
    return jnp.pad(tiled, ((0, 0), (0, pad)))


def kernel(x_nchw, s1, t1, e1, g1, be1, s2, t2, e2, g2, be2, w_fc_rhc,
           b_fc_pad, w1_hwio, b1, w2_hwio, b2, g1_raw, be1_raw, g2_raw,
           be2_raw, w_fc, b_fc):
    n = x_nchw.shape[0]
    bsz = 256
    while n % bsz:
        bsz //= 2
    nblk = n // bsz
    f32 = jnp.float32

    x = x_nchw.reshape(n, H1, W1).astype(jnp.bfloat16)
    xp = jnp.pad(x, ((0, 0), (2, 2), (2, 2)))
    xl = xp.reshape(n, 8, LX).transpose(1, 0, 2)

    s1r = s1.transpose(0, 2, 1, 3).reshape(5, 32, 2, 18, C1)[:, :, :, 2:16, :]
    s1r = jnp.pad(s1r.reshape(5, 32, 2, 14 * C1),
                  ((0, 0), (0, 0), (0, 0), (0, CH1 - 14 * C1)))
    s1r = s1r.reshape(5 * 32, 2 * CH1)
    zrb = [jnp.zeros((32 * k, 2 * CH1), f32) for k in range(4)]
    s1wf = jnp.concatenate(
        [jnp.concatenate([zrb[rb], s1r, zrb[3 - rb]], axis=0)
         for rb in range(4)], axis=1)
    s1w = s1wf.astype(jnp.bfloat16)

    s2r = s2.transpose(0, 2, 1, 3)
    s2r = s2r.reshape(5, 18, C1, 2, 224)[:, 2:16]
    s2r = jnp.pad(s2r.reshape(5, 14 * C1, 2, 224),
                  ((0, 0), (0, CH2 - 14 * C1), (0, 0), (0, CQ2 - 224)))
    s2w = s2r.reshape(K2, N2).astype(jnp.bfloat16)

    wfc = jnp.pad(w_fc_rhc, ((0, 0), (0, CQ2 - 224), (0, 0)))
    wfc = wfc.reshape(FC_K, FC_PAD)

    mm1, gram, rsum = _conv1_call(xl, s1w, n, nblk, bsz)

    s_lanes = (rsum @ s1wf).reshape(8, CH1).sum(axis=0)
    q_lanes = (s1wf * (gram @ s1wf)).sum(axis=0).reshape(8, CH1).sum(axis=0)
    s_ch = s_lanes.reshape(16, C1).sum(axis=0).reshape(1, C1)
    q_ch = q_lanes.reshape(16, C1).sum(axis=0).reshape(1, C1)
    inv1 = 1.0 / (n * H1 * W1)
    mean1 = s_ch * inv1
    var1 = q_ch * inv1 - mean1 * mean1
    scale1 = g1 * jax.lax.rsqrt(var1 + EPS)
    shift1 = be1 - scale1 * mean1
    scsh1 = _minmax_trip(scale1, shift1, 16, CH2)

    mm2, st2 = _conv2_call(mm1, scsh1, s2w, n, nblk, bsz)

    S2 = st2[:, 0:CQ2] + st2[:, CQ2:]
    ch2 = S2.reshape(2, 8, C2).sum(axis=1)
    inv2 = 1.0 / (n * H2 * W2)
    mean2 = ch2[0:1] * inv2
    var2 = ch2[1:2] * inv2 - mean2 * mean2
    scale2 = g2 * jax.lax.rsqrt(var2 + EPS)
    shift2 = be2 - scale2 * mean2
    scsh2 = _minmax_trip(scale2, shift2, 8, CQ2)

    logits = _apply2_call(mm2, scsh2, wfc, b_fc_pad, n, nblk, bsz)
    return logits[:, :NUM_CLASSES]


# speedup vs baseline: 57.8071x; 1.0294x over previous
import functools

import jax
import jax.numpy as jnp
from jax.experimental import pallas as pl
from jax.experimental.pallas import tpu as pltpu

EPS = 1e-5
NUM_CLASSES = 10
H1 = W1 = 28
C1, C2 = 16, 32
H2 = W2 = 14
PR = 7
LX = 128
K1 = 2 * LX
CH1 = 256
N1 = 8 * CH1
CH2 = 256
K2 = 5 * CH2
CQ2 = 256
N2 = 2 * CQ2
FC_PAD = 128
FC_K = PR * CQ2


def _conv1_kernel(xl_ref, s1_ref, mm_ref, g_ref, rs_ref, *, bsz):
    t = pl.program_id(0)
    xl = xl_ref[...]
    lhs = jnp.concatenate([xl[0:PR], xl[1:PR + 1]],
                          axis=2).reshape(PR * bsz, K1)
    p = jnp.dot(lhs, s1_ref[...], preferred_element_type=jnp.float32)
    gram = jax.lax.dot_general(lhs, lhs, (((0,), (0,)), ((), ())),
                               preferred_element_type=jnp.float32)
    rsum = jnp.sum(lhs.astype(jnp.float32), axis=0, keepdims=True)

    @pl.when(t == 0)
    def _init():
        g_ref[...] = jnp.zeros_like(g_ref)
        rs_ref[...] = jnp.zeros_like(rs_ref)

    g_ref[...] += gram
    rs_ref[...] += rsum

    for par in range(2):
        c0 = p[:, (4 * par + 0) * CH1:(4 * par + 1) * CH1]
        c1 = p[:, (4 * par + 1) * CH1:(4 * par + 2) * CH1]
        c2 = p[:, (4 * par + 2) * CH1:(4 * par + 3) * CH1]
        c3 = p[:, (4 * par + 3) * CH1:(4 * par + 4) * CH1]
        pmax = jnp.maximum(jnp.maximum(c0, c1), jnp.maximum(c2, c3))
        pmin = jnp.minimum(jnp.minimum(c0, c1), jnp.minimum(c2, c3))
        mm_ref[2 * par] = pmax.astype(jnp.bfloat16).reshape(PR, bsz, CH1)
        mm_ref[2 * par + 1] = pmin.astype(jnp.bfloat16).reshape(PR, bsz, CH1)


def _conv2_kernel(mm_ref, scsh1_ref, s2_ref, mm2_ref, st_ref, *, bsz):
    t = pl.program_id(0)
    scp = scsh1_ref[0:1, :].reshape(1, 1, CH2)
    scn = scsh1_ref[1:2, :].reshape(1, 1, CH2)
    sh = scsh1_ref[2:3, :].reshape(1, 1, CH2)
    acts = []
    for par in range(2):
        pmax = mm_ref[2 * par].astype(jnp.float32)
        pmin = mm_ref[2 * par + 1].astype(jnp.float32)
        act = jnp.maximum(scp * pmax + scn * pmin + sh, 0.0)
        acts.append(act.astype(jnp.bfloat16))
    zrow = jnp.zeros((1, bsz, 2 * CH2), jnp.bfloat16)
    eo = jnp.concatenate(
        [zrow, jnp.concatenate(acts, axis=2), zrow], axis=0)
    s2 = s2_ref[...]
    lhs0 = jnp.concatenate(
        [eo[0:PR], eo[1:PR + 1], eo[2:PR + 2, :, 0:CH2]],
        axis=2).reshape(PR * bsz, K2)
    lhs1 = jnp.concatenate(
        [eo[0:PR, :, CH2:], eo[1:PR + 1], eo[2:PR + 2]],
        axis=2).reshape(PR * bsz, K2)
    p0 = jnp.dot(lhs0, s2, preferred_element_type=jnp.float32)
    p1 = jnp.dot(lhs1, s2, preferred_element_type=jnp.float32)

    tot = (jnp.sum(p0, axis=0, keepdims=True)
           + jnp.sum(p1, axis=0, keepdims=True))
    ssq = (jnp.sum(p0 * p0, axis=0, keepdims=True)
           + jnp.sum(p1 * p1, axis=0, keepdims=True))

    @pl.when(t == 0)
    def _init():
        st_ref[...] = jnp.zeros_like(st_ref)

    st_ref[0:1, :] += tot
    st_ref[1:2, :] += ssq

    pmax2 = jnp.maximum(jnp.maximum(p0[:, :CQ2], p0[:, CQ2:]),
                        jnp.maximum(p1[:, :CQ2], p1[:, CQ2:]))
    pmin2 = jnp.minimum(jnp.minimum(p0[:, :CQ2], p0[:, CQ2:]),
                        jnp.minimum(p1[:, :CQ2], p1[:, CQ2:]))
    mm2_ref[0] = pmax2.astype(jnp.bfloat16).reshape(PR, bsz, CQ2)
    mm2_ref[1] = pmin2.astype(jnp.bfloat16).reshape(PR, bsz, CQ2)


def _apply2_kernel(mm2_ref, scsh2_ref, wfc_ref, bfc_ref, o_ref, *, bsz):
    scp = scsh2_ref[0:1, :].reshape(1, 1, CQ2)
    scn = scsh2_ref[1:2, :].reshape(1, 1, CQ2)
    sh = scsh2_ref[2:3, :].reshape(1, 1, CQ2)
    pmax = mm2_ref[0].astype(jnp.float32)
    pmin = mm2_ref[1].astype(jnp.float32)
    act = jnp.maximum(scp * pmax + scn * pmin + sh, 0.0)
    lhs = jnp.concatenate([act[h] for h in range(PR)], axis=1)
    o_ref[...] = (jnp.dot(lhs, wfc_ref[...],
                          preferred_element_type=jnp.float32)
                  + bfc_ref[...])


def _conv1_call(xl, s1w, n, nblk, bsz):
    return pl.pallas_call(
        functools.partial(_conv1_kernel, bsz=bsz),
        grid=(nblk,),
        in_specs=[
            pl.BlockSpec((8, bsz, LX), lambda t: (0, t, 0)),
            pl.BlockSpec((K1, N1), lambda t: (0, 0)),
        ],
        out_specs=[
            pl.BlockSpec((4, PR, bsz, CH1), lambda t: (0, 0, t, 0)),
            pl.BlockSpec((K1, K1), lambda t: (0, 0)),
            pl.BlockSpec((1, K1), lambda t: (0, 0)),
        ],
        out_shape=[
            jax.ShapeDtypeStruct((4, PR, n, CH1), jnp.bfloat16),
            jax.ShapeDtypeStruct((K1, K1), jnp.float32),
            jax.ShapeDtypeStruct((1, K1), jnp.float32),
        ],
        compiler_params=pltpu.CompilerParams(
            dimension_semantics=("arbitrary",)),
    )(xl, s1w)


def _conv2_call(mm1, scsh1, s2w, n, nblk, bsz):
    return pl.pallas_call(
        functools.partial(_conv2_kernel, bsz=bsz),
        grid=(nblk,),
        in_specs=[
            pl.BlockSpec((4, PR, bsz, CH1), lambda t: (0, 0, t, 0)),
            pl.BlockSpec((3, CH2), lambda t: (0, 0)),
            pl.BlockSpec((K2, N2), lambda t: (0, 0)),
        ],
        out_specs=[
            pl.BlockSpec((2, PR, bsz, CQ2), lambda t: (0, 0, t, 0)),
            pl.BlockSpec((2, N2), lambda t: (0, 0)),
        ],
        out_shape=[
            jax.ShapeDtypeStruct((2, PR, n, CQ2), jnp.bfloat16),
            jax.ShapeDtypeStruct((2, N2), jnp.float32),
        ],
        compiler_params=pltpu.CompilerParams(
            dimension_semantics=("arbitrary",)),
    )(mm1, scsh1, s2w)


def _apply2_call(mm2, scsh2, wfc, bfc, n, nblk, bsz):
    return pl.pallas_call(
        functools.partial(_apply2_kernel, bsz=bsz),
        grid=(nblk,),
        in_specs=[
            pl.BlockSpec((2, PR, bsz, CQ2), lambda t: (0, 0, t, 0)),
            pl.BlockSpec((3, CQ2), lambda t: (0, 0)),
            pl.BlockSpec((FC_K, FC_PAD), lambda t: (0, 0)),
            pl.BlockSpec((1, FC_PAD), lambda t: (0, 0)),
        ],
        out_specs=pl.BlockSpec((bsz, FC_PAD), lambda t: (t, 0)),
        out_shape=jax.ShapeDtypeStruct((n, FC_PAD), jnp.float32),
        compiler_params=pltpu.CompilerParams(
            dimension_semantics=("arbitrary",)),
    )(mm2, scsh2, wfc, bfc)


def _minmax_trip(scale, shift, reps, lanes):
    trip = jnp.concatenate(
        [jnp.maximum(scale, 0.0), jnp.minimum(scale, 0.0), shift], axis=0)
    tiled = jnp.tile(trip, (1, reps))
    pad = lanes - tiled.shape[1]
    return jnp.pad(tiled, ((0, 0), (0, pad)))


def kernel(x_nchw, s1, t1, e1, g1, be1, s2, t2, e2, g2, be2, w_fc_rhc,
           b_fc_pad, w1_hwio, b1, w2_hwio, b2, g1_raw, be1_raw, g2_raw,
           be2_raw, w_fc, b_fc):
    n = x_nchw.shape[0]
    bsz = 256
    while n % bsz:
        bsz //= 2
    nblk = n // bsz
    bsza = 2 * bsz if n % (2 * bsz) == 0 else bsz
    nblka = n // bsza
    f32 = jnp.float32

    x = x_nchw.reshape(n, H1, W1).astype(jnp.bfloat16)
    xp = jnp.pad(x, ((0, 0), (2, 2), (2, 2)))
    xl = xp.reshape(n, 8, LX).transpose(1, 0, 2)

    s1r = s1.transpose(0, 2, 1, 3).reshape(5, 32, 2, 18, C1)[:, :, :, 2:16, :]
    s1r = jnp.pad(s1r.reshape(5, 32, 2, 14 * C1),
                  ((0, 0), (0, 0), (0, 0), (0, CH1 - 14 * C1)))
    s1r = s1r.reshape(5 * 32, 2 * CH1)
    zrb = [jnp.zeros((32 * k, 2 * CH1), f32) for k in range(4)]
    s1wf = jnp.concatenate(
        [jnp.concatenate([zrb[rb], s1r, zrb[3 - rb]], axis=0)
         for rb in range(4)], axis=1)
    s1w = s1wf.astype(jnp.bfloat16)

    s2r = s2.transpose(0, 2, 1, 3)
    s2r = s2r.reshape(5, 18, C1, 2, 224)[:, 2:16]
    s2r = jnp.pad(s2r.reshape(5, 14 * C1, 2, 224),
                  ((0, 0), (0, CH2 - 14 * C1), (0, 0), (0, CQ2 - 224)))
    s2w = s2r.reshape(K2, N2).astype(jnp.bfloat16)

    wfc = jnp.pad(w_fc_rhc, ((0, 0), (0, CQ2 - 224), (0, 0)))
    wfc = wfc.reshape(FC_K, FC_PAD)

    mm1, gram, rsum = _conv1_call(xl, s1w, n, nblka, bsza)

    s_lanes = (rsum @ s1wf).reshape(8, CH1).sum(axis=0)
    q_lanes = (s1wf * (gram @ s1wf)).sum(axis=0).reshape(8, CH1).sum(axis=0)
    s_ch = s_lanes.reshape(16, C1).sum(axis=0).reshape(1, C1)
    q_ch = q_lanes.reshape(16, C1).sum(axis=0).reshape(1, C1)
    inv1 = 1.0 / (n * H1 * W1)
    mean1 = s_ch * inv1
    var1 = q_ch * inv1 - mean1 * mean1
    scale1 = g1 * jax.lax.rsqrt(var1 + EPS)
    shift1 = be1 - scale1 * mean1
    scsh1 = _minmax_trip(scale1, shift1, 16, CH2)

    mm2, st2 = _conv2_call(mm1, scsh1, s2w, n, nblk, bsz)

    S2 = st2[:, 0:CQ2] + st2[:, CQ2:]
    ch2 = S2.reshape(2, 8, C2).sum(axis=1)
    inv2 = 1.0 / (n * H2 * W2)
    mean2 = ch2[0:1] * inv2
    var2 = ch2[1:2] * inv2 - mean2 * mean2
    scale2 = g2 * jax.lax.rsqrt(var2 + EPS)
    shift2 = be2 - scale2 * mean2
    scsh2 = _minmax_trip(scale2, shift2, 8, CQ2)

    logits = _apply2_call(mm2, scsh2, wfc, b_fc_pad, n, nblka, bsza)
    return logits[:, :NUM_CLASSES]
